# Initial kernel scaffold; baseline (speedup 1.0000x reference)
import jax
import jax.numpy as jnp
from jax.experimental import pallas as pl


def kernel(user_emb, item_emb, adj_indices_0, adj_values_0, adj_indices_1, adj_values_1, W1, W2):
    raise NotImplementedError("write your pallas kernel here")


# speedup vs baseline: 4.6563x; 4.6563x over previous
import functools

import jax
import jax.numpy as jnp
from jax import lax
from jax.experimental import pallas as pl
from jax.experimental.pallas import tpu as pltpu
from jax.experimental.pallas import tpu_sc as plsc

USER_NUM = 30000
ITEM_NUM = 20000
N = USER_NUM + ITEM_NUM
EMB = 64
E = 800000
N_LAYERS = 3

NP_ = 50176
HALF = NP_ // 2
STRIPE = HALF // 16
PT = E // 16
K = 2000
NCHUNK = PT // K
W = 256
CAP = 2064
LANES = 16

_mesh = plsc.VectorSubcoreMesh(
    core_axis_name="c", subcore_axis_name="s", num_cores=2, num_subcores=16
)


def _spmm_body(x_hbm, rows_hbm, cols_hbm, vals_hbm, y_hbm,
               rows_v, cols_v, vals_v, cb_row, cb_col, cb_val, widx, gbuf, acc):
    cid = lax.axis_index("c")
    sid = lax.axis_index("s")
    base = cid * HALF
    ebase = sid * PT
    zeros16 = jnp.zeros((LANES,), jnp.float32)
    lane = lax.iota(jnp.int32, LANES)

    if True:
        @pl.loop(0, W)
        def _(r):
            for q in range(4):
                gbuf[r, pl.ds(16 * q, 16)] = zeros16

        for z in range(STRIPE // W):
            pltpu.sync_copy(gbuf, acc.at[pl.ds(sid * STRIPE + z * W, W)])
        rem = STRIPE % W
        if rem:
            pltpu.sync_copy(gbuf.at[pl.ds(0, rem)],
                            acc.at[pl.ds(sid * STRIPE + (STRIPE // W) * W, rem)])

        @pl.loop(0, CAP // 16)
        def _(j):
            safe = (j * 16 + lane) & 16383
            cb_row[pl.ds(j * 16, 16)] = safe
            cb_col[pl.ds(j * 16, 16)] = safe
            cb_val[pl.ds(j * 16, 16)] = zeros16

        plsc.subcore_barrier()

        @pl.loop(0, NCHUNK)
        def _(c):
            off = ebase + c * K
            pltpu.sync_copy(rows_hbm.at[pl.ds(off, K)], rows_v)
            pltpu.sync_copy(cols_hbm.at[pl.ds(off, K)], cols_v)
            pltpu.sync_copy(vals_hbm.at[pl.ds(off, K)], vals_v)

            def compact(j, pos):
                rv = rows_v[pl.ds(16 * j, 16)]
                m = (rv >= base) & (rv < base + HALF)
                plsc.store_compressed(cb_row.at[pl.ds(pos, 16)], rv - base, mask=m)
                plsc.store_compressed(cb_col.at[pl.ds(pos, 16)],
                                      cols_v[pl.ds(16 * j, 16)], mask=m)
                plsc.store_compressed(cb_val.at[pl.ds(pos, 16)],
                                      vals_v[pl.ds(16 * j, 16)], mask=m)
                return pos + jnp.max(plsc.all_reduce_population_count(m))

            n = lax.fori_loop(0, K // 16, compact, jnp.int32(0))

            nf = (n // 16) * 16
            v = cb_val[pl.ds(nf, 16)]
            cb_val[pl.ds(nf, 16)] = jnp.where(lane < (n - nf), v, 0.0)
            t = (n + W - 1) // W

            def ztail(k, _):
                cb_val[pl.ds(nf + 16 + 16 * k, 16)] = zeros16
                return _

            lax.fori_loop(0, (t * W - nf - 16) // 16, ztail, 0)

            def window(w, _):
                for m16 in range(W // 16):
                    widx[pl.ds(16 * m16, 16)] = cb_row[pl.ds(w * W + 16 * m16, 16)]
                pltpu.sync_copy(x_hbm.at[cb_col.at[pl.ds(w * W, W)]], gbuf)

                def scale(e, _2):
                    vs = plsc.load_gather(
                        cb_val, [jnp.full((LANES,), w * W + e, jnp.int32)])
                    for q in range(4):
                        gbuf[e, pl.ds(16 * q, 16)] = gbuf[e, pl.ds(16 * q, 16)] * vs
                    return _2

                lax.fori_loop(0, W, scale, 0)
                pltpu.sync_copy(gbuf, acc.at[widx], add=True)
                return _

            lax.fori_loop(0, t, window, 0)

        plsc.subcore_barrier()
        pltpu.sync_copy(acc.at[pl.ds(sid * STRIPE, STRIPE)],
                        y_hbm.at[pl.ds(base + sid * STRIPE, STRIPE)])


@functools.partial(
    pl.kernel,
    out_type=jax.ShapeDtypeStruct((NP_, EMB), jnp.float32),
    mesh=_mesh,
    compiler_params=pltpu.CompilerParams(
        needs_layout_passes=False, use_tc_tiling_on_sc=False),
    scratch_types=[
        pltpu.VMEM((K,), jnp.int32),
        pltpu.VMEM((K,), jnp.int32),
        pltpu.VMEM((K,), jnp.float32),
        pltpu.VMEM((CAP,), jnp.int32),
        pltpu.VMEM((CAP,), jnp.int32),
        pltpu.VMEM((CAP,), jnp.float32),
        pltpu.VMEM((W,), jnp.int32),
        pltpu.VMEM((W, EMB), jnp.float32),
        pltpu.VMEM_SHARED((HALF, EMB), jnp.float32),
    ],
)
def _spmm_step(x_hbm, rows_hbm, cols_hbm, vals_hbm, y_hbm, *scratch):
    _spmm_body(x_hbm, rows_hbm, cols_hbm, vals_hbm, y_hbm, *scratch)


_RB = 1792
_GRID = NP_ // _RB


def _att_kernel(x0, a1, a2, a3, b1, b2, b3, w1, w2, o0, o1):
    m0 = (x0[...] + a1[...] + a2[...] + a3[...]) * 0.25
    m1 = (x0[...] + b1[...] + b2[...] + b3[...]) * 0.25
    outs = (o0, o1)
    for i in range(2):
        w1i = w1[i]
        w2i = w2[i]
        h0 = jnp.tanh(jnp.dot(m0, w1i, preferred_element_type=jnp.float32))
        h1 = jnp.tanh(jnp.dot(m1, w1i, preferred_element_type=jnp.float32))
        s0 = jnp.sum(h0 * w2i[None, :], axis=1, keepdims=True)
        s1 = jnp.sum(h1 * w2i[None, :], axis=1, keepdims=True)
        mx = jnp.maximum(s0, s1)
        e0 = jnp.exp(s0 - mx)
        e1 = jnp.exp(s1 - mx)
        a0 = e0 / (e0 + e1)
        outs[i][...] = a0 * m0 + (1.0 - a0) * m1


def _attention(x0, ys0, ys1, W1, W2):
    row_spec = pl.BlockSpec((_RB, EMB), lambda i: (i, 0))
    return pl.pallas_call(
        _att_kernel,
        grid=(_GRID,),
        in_specs=[row_spec] * 7 + [
            pl.BlockSpec((2, EMB, EMB // 4), lambda i: (0, 0, 0)),
            pl.BlockSpec((2, EMB // 4), lambda i: (0, 0)),
        ],
        out_specs=[row_spec, row_spec],
        out_shape=[jax.ShapeDtypeStruct((NP_, EMB), jnp.float32)] * 2,
    )(x0, ys0[0], ys0[1], ys0[2], ys1[0], ys1[1], ys1[2], W1, W2)


def kernel(user_emb, item_emb, adj_indices_0, adj_values_0,
           adj_indices_1, adj_values_1, W1, W2):
    ego = jnp.concatenate([user_emb, item_emb], axis=0)
    x0 = jnp.pad(ego, ((0, NP_ - N), (0, 0)))

    def propagate(indices, values):
        rows = indices[0].astype(jnp.int32)
        cols = indices[1].astype(jnp.int32)
        vals = values.astype(jnp.float32)
        ys = []
        x = x0
        for _ in range(N_LAYERS):
            x = _spmm_step(x, rows, cols, vals)
            ys.append(x)
        return ys

    ys0 = propagate(adj_indices_0, adj_values_0)
    ys1 = propagate(adj_indices_1, adj_values_1)

    o0, o1 = _attention(x0, ys0, ys1, W1, jnp.squeeze(W2, axis=2))
    user_beh = jnp.stack([o0[:USER_NUM], o1[:USER_NUM]], axis=0)
    item_beh = jnp.stack([o0[USER_NUM:N], o1[USER_NUM:N]], axis=0)
    return (user_beh, item_beh)
